# baseline (speedup 1.0000x reference)
import functools

import jax
import jax.numpy as jnp
from jax import lax
from jax.experimental import pallas as pl
from jax.experimental.pallas import tpu as pltpu
from jax.experimental.pallas import tpu_sc as plsc

_LANES = 16
_NW = 32


def _sad_guess_sc(n_coeffs, n_atoms, chunk_atoms):
    per_atom = n_coeffs // n_atoms
    apw = n_atoms // _NW
    n_chunks = apw // chunk_atoms
    cw = chunk_atoms * per_atom
    zs = chunk_atoms + 8 - (chunk_atoms + 8 - n_atoms) % 8
    assert (n_atoms - zs) % 8 == 0 and zs >= chunk_atoms + 7

    mesh = plsc.VectorSubcoreMesh(core_axis_name="c", subcore_axis_name="s")

    @functools.partial(
        pl.kernel,
        out_type=jax.ShapeDtypeStruct((n_coeffs,), jnp.float32),
        mesh=mesh,
        compiler_params=pltpu.CompilerParams(needs_layout_passes=False),
        scratch_types=[
            pltpu.VMEM((2048,), jnp.float32),
            pltpu.VMEM((2048,), jnp.float32),
            [pltpu.VMEM((cw,), jnp.int32) for _ in range(2)],
            [pltpu.VMEM((cw,), jnp.float32) for _ in range(2)],
            [pltpu.VMEM((zs + 16,), jnp.float32) for _ in range(2)],
            [pltpu.VMEM((cw,), jnp.float32) for _ in range(2)],
            [pltpu.SemaphoreType.DMA for _ in range(2)],
            [pltpu.SemaphoreType.DMA for _ in range(2)],
            pltpu.SemaphoreType.DMA,
        ],
    )
    def k(bfi_hbm, dual_hbm, z_hbm, sad_hbm, std_hbm, out_hbm,
          sad_v, std_v, bfi_v, dual_v, z_v, out_v, in_sem, out_sem, tab_sem):
        wid = lax.axis_index("s") * 2 + lax.axis_index("c")
        t1 = pltpu.async_copy(sad_hbm, sad_v, tab_sem)
        t2 = pltpu.async_copy(std_hbm, std_v, tab_sem)
        atom_base_w = wid * apw

        def start_in(ci, buf):
            atom_base = atom_base_w + ci * chunk_atoms
            cbase = atom_base * per_atom
            zoff = jnp.minimum((atom_base // 8) * 8, n_atoms - zs)
            return (
                pltpu.async_copy(bfi_hbm.at[pl.ds(cbase, cw)], bfi_v[buf],
                                 in_sem[buf]),
                pltpu.async_copy(dual_hbm.at[pl.ds(cbase, cw)], dual_v[buf],
                                 in_sem[buf]),
                pltpu.async_copy(z_hbm.at[pl.ds(zoff, zs)],
                                 z_v[buf].at[pl.ds(0, zs)], in_sem[buf]),
                atom_base - zoff,
            )

        pend_in = {0: start_in(0, 0)}
        pend_out = {}
        t1.wait()
        t2.wait()
        for ci in range(n_chunks):
            buf = ci % 2
            if ci + 1 < n_chunks:
                pend_in[ci + 1] = start_in(ci + 1, 1 - buf)
            c1, c2, c3, zlocal = pend_in.pop(ci)
            c1.wait()
            c2.wait()
            c3.wait()
            if ci >= 2:
                pend_out.pop(ci - 2).wait()
            bfi_b, dual_b, z_b, out_b = bfi_v[buf], dual_v[buf], z_v[buf], out_v[buf]

            @plsc.parallel_loop(0, chunk_atoms // 5, 1, unroll=2)
            def group_body(gg):
                a0 = gg * 5
                z16 = z_b[pl.ds(zlocal + a0, _LANES)]
                for j in range(5):
                    a = a0 + j
                    b = bfi_b[pl.ds(a * per_atom, _LANES)]
                    g = plsc.load_gather(sad_v, [b])
                    s = plsc.load_gather(std_v, [b])
                    du = dual_b[pl.ds(a * per_atom, _LANES)]
                    d = s * s * du
                    ne = jnp.sum(du * g)
                    den = jnp.sum(d * du)
                    num = z16[j] - ne
                    f = jnp.full((_LANES,), num, jnp.float32) / jnp.full(
                        (_LANES,), den, jnp.float32)
                    out_b[pl.ds(a * per_atom, _LANES)] = g + d * f

            cbase = (atom_base_w + ci * chunk_atoms) * per_atom
            pend_out[ci] = pltpu.async_copy(
                out_b, out_hbm.at[pl.ds(cbase, cw)], out_sem[buf])
        for ci in sorted(pend_out):
            pend_out.pop(ci).wait()

    return k


def kernel(basis_function_ind, dual_basis_integrals, coeff_ind_to_node_ind,
           atomic_numbers, sad_coeffs_per_basis_func, coeff_std_per_basis_func):
    del coeff_ind_to_node_ind
    n_coeffs = basis_function_ind.shape[0]
    n_atoms = atomic_numbers.shape[0]
    k = _sad_guess_sc(n_coeffs, n_atoms, chunk_atoms=625)
    return k(basis_function_ind, dual_basis_integrals, atomic_numbers,
             sad_coeffs_per_basis_func, coeff_std_per_basis_func)


# speedup vs baseline: 1.2006x; 1.2006x over previous
import functools

import jax
import jax.numpy as jnp
from jax import lax
from jax.experimental import pallas as pl
from jax.experimental.pallas import tpu as pltpu
from jax.experimental.pallas import tpu_sc as plsc

_LANES = 16
_NW = 32


def _zwin(chunk_atoms, n_atoms):
    zs = chunk_atoms + 7
    zs += (n_atoms - zs) % 8
    assert (n_atoms - zs) % 8 == 0 and zs >= chunk_atoms + 7
    return zs


def _sad_guess_sc(n_coeffs, n_atoms, chunk_sizes):
    per_atom = n_coeffs // n_atoms
    apw = n_atoms // _NW
    assert sum(chunk_sizes) == apw
    max_chunk = max(chunk_sizes)
    cw = max_chunk * per_atom
    zwins = [_zwin(c, n_atoms) for c in chunk_sizes]
    zs_max = max(zwins)
    n_chunks = len(chunk_sizes)
    starts = [sum(chunk_sizes[:i]) for i in range(n_chunks)]

    mesh = plsc.VectorSubcoreMesh(core_axis_name="c", subcore_axis_name="s")

    @functools.partial(
        pl.kernel,
        out_type=jax.ShapeDtypeStruct((n_coeffs,), jnp.float32),
        mesh=mesh,
        compiler_params=pltpu.CompilerParams(needs_layout_passes=False),
        scratch_types=[
            pltpu.VMEM((2048,), jnp.float32),
            pltpu.VMEM((2048,), jnp.float32),
            [pltpu.VMEM((cw,), jnp.int32) for _ in range(2)],
            [pltpu.VMEM((cw,), jnp.float32) for _ in range(2)],
            [pltpu.VMEM((zs_max,), jnp.float32) for _ in range(2)],
            [pltpu.VMEM((cw,), jnp.float32) for _ in range(2)],
            [pltpu.SemaphoreType.DMA for _ in range(2)],
            [pltpu.SemaphoreType.DMA for _ in range(2)],
            pltpu.SemaphoreType.DMA,
        ],
    )
    def k(bfi_hbm, dual_hbm, z_hbm, sad_hbm, std_hbm, out_hbm,
          sad_v, std_v, bfi_v, dual_v, z_v, out_v, in_sem, out_sem, tab_sem):
        wid = lax.axis_index("s") * 2 + lax.axis_index("c")
        t1 = pltpu.async_copy(sad_hbm, sad_v, tab_sem)
        t2 = pltpu.async_copy(std_hbm, std_v, tab_sem)
        atom_base_w = wid * apw

        def start_in(ci, buf):
            na, nz = chunk_sizes[ci], zwins[ci]
            atom_base = atom_base_w + starts[ci]
            cbase = atom_base * per_atom
            zoff = jnp.minimum((atom_base // 8) * 8, n_atoms - nz)
            nw = na * per_atom
            return (
                pltpu.async_copy(bfi_hbm.at[pl.ds(cbase, nw)],
                                 bfi_v[buf].at[pl.ds(0, nw)], in_sem[buf]),
                pltpu.async_copy(dual_hbm.at[pl.ds(cbase, nw)],
                                 dual_v[buf].at[pl.ds(0, nw)], in_sem[buf]),
                pltpu.async_copy(z_hbm.at[pl.ds(zoff, nz)],
                                 z_v[buf].at[pl.ds(0, nz)], in_sem[buf]),
                atom_base - zoff,
            )

        pend_in = {0: start_in(0, 0)}
        pend_out = {}
        t1.wait()
        t2.wait()
        for ci in range(n_chunks):
            buf = ci % 2
            if ci + 1 < n_chunks:
                pend_in[ci + 1] = start_in(ci + 1, 1 - buf)
            c1, c2, c3, zlocal = pend_in.pop(ci)
            c1.wait()
            c2.wait()
            c3.wait()
            if ci >= 2:
                pend_out.pop(ci - 2).wait()
            bfi_b, dual_b, z_b, out_b = (bfi_v[buf], dual_v[buf], z_v[buf],
                                         out_v[buf])

            @plsc.parallel_loop(0, chunk_sizes[ci], 1, unroll=4)
            def atom_body(a):
                b = bfi_b[pl.ds(a * per_atom, _LANES)]
                g = plsc.load_gather(sad_v, [b])
                s = plsc.load_gather(std_v, [b])
                du = dual_b[pl.ds(a * per_atom, _LANES)]
                d = s * s * du
                ne = jnp.sum(du * g)
                den = jnp.sum(d * du)
                zq = plsc.load_gather(
                    z_b, [jnp.full((_LANES,), zlocal + a, jnp.int32)])
                f = (zq - ne) / den
                out_b[pl.ds(a * per_atom, _LANES)] = g + d * f

            nw = chunk_sizes[ci] * per_atom
            cbase = (atom_base_w + starts[ci]) * per_atom
            pend_out[ci] = pltpu.async_copy(
                out_b.at[pl.ds(0, nw)], out_hbm.at[pl.ds(cbase, nw)],
                out_sem[buf])
        for ci in sorted(pend_out):
            pend_out.pop(ci).wait()

    return k


def kernel(basis_function_ind, dual_basis_integrals, coeff_ind_to_node_ind,
           atomic_numbers, sad_coeffs_per_basis_func, coeff_std_per_basis_func):
    del coeff_ind_to_node_ind
    n_coeffs = basis_function_ind.shape[0]
    n_atoms = atomic_numbers.shape[0]
    k = _sad_guess_sc(n_coeffs, n_atoms,
                      chunk_sizes=(125, 775, 775, 775, 675))
    return k(basis_function_ind, dual_basis_integrals, atomic_numbers,
             sad_coeffs_per_basis_func, coeff_std_per_basis_func)


# speedup vs baseline: 1.2470x; 1.0386x over previous
import functools

import jax
import jax.numpy as jnp
from jax import lax
from jax.experimental import pallas as pl
from jax.experimental.pallas import tpu as pltpu
from jax.experimental.pallas import tpu_sc as plsc

_LANES = 16
_NW = 32


def _sad_guess_sc(n_coeffs, n_atoms, chunk_atoms):
    per_atom = n_coeffs // n_atoms
    apw = n_atoms // _NW
    n_chunks = apw // chunk_atoms
    cw = chunk_atoms * per_atom
    zs = chunk_atoms + 8 - (chunk_atoms + 8 - n_atoms) % 8
    assert (n_atoms - zs) % 8 == 0 and zs >= chunk_atoms + 7

    mesh = plsc.VectorSubcoreMesh(core_axis_name="c", subcore_axis_name="s")

    @functools.partial(
        pl.kernel,
        out_type=jax.ShapeDtypeStruct((n_coeffs,), jnp.float32),
        mesh=mesh,
        compiler_params=pltpu.CompilerParams(needs_layout_passes=False),
        scratch_types=[
            pltpu.VMEM((2048,), jnp.float32),
            pltpu.VMEM((2048,), jnp.float32),
            [pltpu.VMEM((cw,), jnp.int32) for _ in range(2)],
            [pltpu.VMEM((cw,), jnp.float32) for _ in range(2)],
            [pltpu.VMEM((zs + 16,), jnp.float32) for _ in range(2)],
            [pltpu.VMEM((cw,), jnp.float32) for _ in range(2)],
            [pltpu.SemaphoreType.DMA for _ in range(2)],
            [pltpu.SemaphoreType.DMA for _ in range(2)],
            pltpu.SemaphoreType.DMA,
        ],
    )
    def k(bfi_hbm, dual_hbm, z_hbm, sad_hbm, std_hbm, out_hbm,
          sad_v, std_v, bfi_v, dual_v, z_v, out_v, in_sem, out_sem, tab_sem):
        wid = lax.axis_index("s") * 2 + lax.axis_index("c")
        t1 = pltpu.async_copy(sad_hbm, sad_v, tab_sem)
        t2 = pltpu.async_copy(std_hbm, std_v, tab_sem)
        atom_base_w = wid * apw

        def start_in(ci, buf):
            atom_base = atom_base_w + ci * chunk_atoms
            cbase = atom_base * per_atom
            zoff = jnp.minimum((atom_base // 8) * 8, n_atoms - zs)
            return (
                pltpu.async_copy(bfi_hbm.at[pl.ds(cbase, cw)], bfi_v[buf],
                                 in_sem[buf]),
                pltpu.async_copy(dual_hbm.at[pl.ds(cbase, cw)], dual_v[buf],
                                 in_sem[buf]),
                pltpu.async_copy(z_hbm.at[pl.ds(zoff, zs)],
                                 z_v[buf].at[pl.ds(0, zs)], in_sem[buf]),
                atom_base - zoff,
            )

        pend_in = {0: start_in(0, 0)}
        pend_out = {}
        t1.wait()
        t2.wait()
        for ci in range(n_chunks):
            buf = ci % 2
            if ci + 1 < n_chunks:
                pend_in[ci + 1] = start_in(ci + 1, 1 - buf)
            c1, c2, c3, zlocal = pend_in.pop(ci)
            c1.wait()
            c2.wait()
            c3.wait()
            if ci >= 2:
                pend_out.pop(ci - 2).wait()
            bfi_b, dual_b, z_b, out_b = bfi_v[buf], dual_v[buf], z_v[buf], out_v[buf]

            @plsc.parallel_loop(0, chunk_atoms, 1, unroll=4)
            def atom_body(a):
                b = bfi_b[pl.ds(a * per_atom, _LANES)]
                g = plsc.load_gather(sad_v, [b])
                s = plsc.load_gather(std_v, [b])
                du = dual_b[pl.ds(a * per_atom, _LANES)]
                d = s * s * du
                ne = jnp.sum(du * g)
                den = jnp.sum(d * du)
                zq = plsc.load_gather(
                    z_b, [jnp.full((_LANES,), zlocal + a, jnp.int32)])
                f = (zq - ne) / den
                out_b[pl.ds(a * per_atom, _LANES)] = g + d * f

            cbase = (atom_base_w + ci * chunk_atoms) * per_atom
            pend_out[ci] = pltpu.async_copy(
                out_b, out_hbm.at[pl.ds(cbase, cw)], out_sem[buf])
        for ci in sorted(pend_out):
            pend_out.pop(ci).wait()

    return k


def kernel(basis_function_ind, dual_basis_integrals, coeff_ind_to_node_ind,
           atomic_numbers, sad_coeffs_per_basis_func, coeff_std_per_basis_func):
    del coeff_ind_to_node_ind
    n_coeffs = basis_function_ind.shape[0]
    n_atoms = atomic_numbers.shape[0]
    k = _sad_guess_sc(n_coeffs, n_atoms, chunk_atoms=625)
    return k(basis_function_ind, dual_basis_integrals, atomic_numbers,
             sad_coeffs_per_basis_func, coeff_std_per_basis_func)


# speedup vs baseline: 1.2852x; 1.0306x over previous
import functools

import jax
import jax.numpy as jnp
from jax import lax
from jax.experimental import pallas as pl
from jax.experimental.pallas import tpu as pltpu
from jax.experimental.pallas import tpu_sc as plsc

_LANES = 16
_NW = 32


def _sad_guess_sc(n_coeffs, n_atoms, chunk_atoms):
    per_atom = n_coeffs // n_atoms
    apw = n_atoms // _NW
    n_chunks = apw // chunk_atoms
    cw = chunk_atoms * per_atom
    zs = chunk_atoms + 8 - (chunk_atoms + 8 - n_atoms) % 8
    assert (n_atoms - zs) % 8 == 0 and zs >= chunk_atoms + 7

    mesh = plsc.VectorSubcoreMesh(core_axis_name="c", subcore_axis_name="s")

    @functools.partial(
        pl.kernel,
        out_type=jax.ShapeDtypeStruct((n_coeffs,), jnp.float32),
        mesh=mesh,
        compiler_params=pltpu.CompilerParams(needs_layout_passes=False),
        scratch_types=[
            pltpu.VMEM((2048,), jnp.float32),
            pltpu.VMEM((2048,), jnp.float32),
            [pltpu.VMEM((cw,), jnp.int32) for _ in range(2)],
            [pltpu.VMEM((cw,), jnp.float32) for _ in range(2)],
            [pltpu.VMEM((zs + 16,), jnp.float32) for _ in range(2)],
            [pltpu.VMEM((cw,), jnp.float32) for _ in range(2)],
            [pltpu.SemaphoreType.DMA for _ in range(2)],
            [pltpu.SemaphoreType.DMA for _ in range(2)],
            pltpu.SemaphoreType.DMA,
        ],
    )
    def k(bfi_hbm, dual_hbm, z_hbm, sad_hbm, std_hbm, out_hbm,
          sad_v, std_v, bfi_v, dual_v, z_v, out_v, in_sem, out_sem, tab_sem):
        wid = lax.axis_index("s") * 2 + lax.axis_index("c")
        t1 = pltpu.async_copy(sad_hbm, sad_v, tab_sem)
        t2 = pltpu.async_copy(std_hbm, std_v, tab_sem)
        atom_base_w = wid * apw

        def start_in(ci, buf):
            atom_base = atom_base_w + ci * chunk_atoms
            cbase = atom_base * per_atom
            zoff = jnp.minimum((atom_base // 8) * 8, n_atoms - zs)
            return (
                pltpu.async_copy(bfi_hbm.at[pl.ds(cbase, cw)], bfi_v[buf],
                                 in_sem[buf]),
                pltpu.async_copy(dual_hbm.at[pl.ds(cbase, cw)], dual_v[buf],
                                 in_sem[buf]),
                pltpu.async_copy(z_hbm.at[pl.ds(zoff, zs)],
                                 z_v[buf].at[pl.ds(0, zs)], in_sem[buf]),
                atom_base - zoff,
            )

        pend_in = {0: start_in(0, 0)}
        pend_out = {}
        t1.wait()
        t2.wait()
        for ci in range(n_chunks):
            buf = ci % 2
            if ci + 1 < n_chunks:
                pend_in[ci + 1] = start_in(ci + 1, 1 - buf)
            c1, c2, c3, zlocal = pend_in.pop(ci)
            c1.wait()
            c2.wait()
            c3.wait()
            if ci >= 2:
                pend_out.pop(ci - 2).wait()
            bfi_b, dual_b, z_b, out_b = bfi_v[buf], dual_v[buf], z_v[buf], out_v[buf]

            @plsc.parallel_loop(0, chunk_atoms, 1, unroll=5)
            def atom_body(a):
                b = bfi_b[pl.ds(a * per_atom, _LANES)]
                g = plsc.load_gather(sad_v, [b])
                s = plsc.load_gather(std_v, [b])
                du = dual_b[pl.ds(a * per_atom, _LANES)]
                d = s * s * du
                ne = jnp.sum(du * g)
                den = jnp.sum(d * du)
                zq = plsc.load_gather(
                    z_b, [jnp.full((_LANES,), zlocal + a, jnp.int32)])
                f = (zq - ne) / den
                out_b[pl.ds(a * per_atom, _LANES)] = g + d * f

            cbase = (atom_base_w + ci * chunk_atoms) * per_atom
            pend_out[ci] = pltpu.async_copy(
                out_b, out_hbm.at[pl.ds(cbase, cw)], out_sem[buf])
        for ci in sorted(pend_out):
            pend_out.pop(ci).wait()

    return k


def kernel(basis_function_ind, dual_basis_integrals, coeff_ind_to_node_ind,
           atomic_numbers, sad_coeffs_per_basis_func, coeff_std_per_basis_func):
    del coeff_ind_to_node_ind
    n_coeffs = basis_function_ind.shape[0]
    n_atoms = atomic_numbers.shape[0]
    k = _sad_guess_sc(n_coeffs, n_atoms, chunk_atoms=625)
    return k(basis_function_ind, dual_basis_integrals, atomic_numbers,
             sad_coeffs_per_basis_func, coeff_std_per_basis_func)
